# baseline (speedup 1.0000x reference)
import numpy as np
import jax
import jax.numpy as jnp
from jax import lax
from jax.experimental import pallas as pl
from jax.experimental.pallas import tpu as pltpu
from jax.experimental.pallas import tpu_sc as plsc

EMBEDDING_DIM = 1024
SEQUENCE_LEN = 160
VOCAB_SIZE = 100000
BATCH = 1024

NUM_CORES = 2
NUM_SUBCORES = 16
NW = NUM_CORES * NUM_SUBCORES
ROWS = BATCH * SEQUENCE_LEN
ROWS_PER_W = ROWS // NW
SEQS_PER_W = BATCH // NW
CHUNK = 16
NCH = ROWS_PER_W // CHUNK
PBLK = 16
LANES = 16
GROUPS = EMBEDDING_DIM // LANES
FACTOR = float(np.sqrt(EMBEDDING_DIM))


def _pos_encoding() -> np.ndarray:
    depth = EMBEDDING_DIM / 2
    positions = np.arange(SEQUENCE_LEN)[:, np.newaxis]
    depths = np.arange(depth)[np.newaxis, :] / depth
    rates = 1 / 10000 ** depths
    radians = positions * rates
    return np.concatenate(
        [np.sin(radians), np.cos(radians)], axis=-1).astype(np.float32)


_PE = _pos_encoding()


def _out_rows() -> np.ndarray:
    w = np.arange(NW)[:, None, None]
    c = np.arange(NCH)[None, :, None]
    j = np.arange(CHUNK)[None, None, :]
    s = (c & 1) * CHUNK + j
    p = c >> 1
    return (w * ROWS_PER_W + s * SEQUENCE_LEN + p).astype(np.int32)


_OUT_ROWS = _out_rows()


def _body(table, idx, pe, orow, out, idx_v, oix_v, pe_v,
          in0, in1, out0, out1, gs0, gs1, ws0, ws1):
    wid = lax.axis_index("s") * NUM_CORES + lax.axis_index("c")
    base = wid * ROWS_PER_W
    pltpu.sync_copy(idx.at[pl.ds(base, ROWS_PER_W)], idx_v)
    pltpu.sync_copy(orow.at[wid], oix_v)

    pltpu.async_copy(table.at[idx_v.at[pl.ds(0, CHUNK)]], in0, gs0)
    pltpu.async_copy(table.at[idx_v.at[pl.ds(CHUNK, CHUNK)]], in1, gs1)

    def step(c, inb, outb, gs, ws):
        pltpu.make_async_copy(table.at[pl.ds(0, CHUNK)], inb, gs).wait()

        @pl.when(c >= 2)
        def _():
            pltpu.make_async_copy(outb, out.at[pl.ds(0, CHUNK)], ws).wait()

        pr = (c >> 1) & (PBLK - 1)

        def grp(g, _):
            sl = pl.ds(g * LANES, LANES)
            peg = pe_v[pr, sl]
            for r in range(CHUNK):
                outb[r, sl] = inb[r, sl] * FACTOR + peg
            return 0
        lax.fori_loop(0, GROUPS, grp, 0, unroll=2)

        pltpu.async_copy(outb, out.at[oix_v.at[c]], ws)

        @pl.when(c < NCH - 2)
        def _():
            pltpu.async_copy(
                table.at[idx_v.at[pl.ds((c + 2) * CHUNK, CHUNK)]], inb, gs)

    def c_loop(c, _):
        @pl.when((c & 31) == 0)
        def _():
            pltpu.sync_copy(pe.at[pl.ds((c >> 5) * PBLK, PBLK)], pe_v)

        @pl.when((c & 1) == 0)
        def _():
            step(c, in0, out0, gs0, ws0)

        @pl.when((c & 1) == 1)
        def _():
            step(c, in1, out1, gs1, ws1)
        return 0

    lax.fori_loop(0, NCH, c_loop, 0)

    pltpu.make_async_copy(out0, out.at[pl.ds(0, CHUNK)], ws0).wait()
    pltpu.make_async_copy(out1, out.at[pl.ds(0, CHUNK)], ws1).wait()


@jax.jit
def _embed(encoding, table):
    idx = (encoding.reshape(NW, SEQS_PER_W, SEQUENCE_LEN)
           .transpose(0, 2, 1).reshape(ROWS).astype(jnp.int32))
    pe = jnp.asarray(_PE)
    orow = jnp.asarray(_OUT_ROWS)
    mesh = plsc.VectorSubcoreMesh(core_axis_name="c", subcore_axis_name="s")
    k = pl.kernel(
        _body,
        out_type=jax.ShapeDtypeStruct((ROWS, EMBEDDING_DIM), jnp.float32),
        mesh=mesh,
        scratch_types=[
            pltpu.VMEM((ROWS_PER_W,), jnp.int32),
            pltpu.VMEM((NCH, CHUNK), jnp.int32),
            pltpu.VMEM((PBLK, EMBEDDING_DIM), jnp.float32),
            pltpu.VMEM((CHUNK, EMBEDDING_DIM), jnp.float32),
            pltpu.VMEM((CHUNK, EMBEDDING_DIM), jnp.float32),
            pltpu.VMEM((CHUNK, EMBEDDING_DIM), jnp.float32),
            pltpu.VMEM((CHUNK, EMBEDDING_DIM), jnp.float32),
            pltpu.SemaphoreType.DMA,
            pltpu.SemaphoreType.DMA,
            pltpu.SemaphoreType.DMA,
            pltpu.SemaphoreType.DMA,
        ],
    )
    out = k(table, idx, pe, orow)
    return out.reshape(BATCH, SEQUENCE_LEN, EMBEDDING_DIM)


def kernel(encoding, table):
    return _embed(encoding, table)


# speedup vs baseline: 1.6855x; 1.6855x over previous
import numpy as np
import jax
import jax.numpy as jnp
from jax import lax
from jax.experimental import pallas as pl
from jax.experimental.pallas import tpu as pltpu
from jax.experimental.pallas import tpu_sc as plsc

EMBEDDING_DIM = 1024
SEQUENCE_LEN = 160
VOCAB_SIZE = 100000
BATCH = 1024

NUM_CORES = 2
NUM_SUBCORES = 16
NW = NUM_CORES * NUM_SUBCORES
ROWS = BATCH * SEQUENCE_LEN
ROWS_PER_W = ROWS // NW
SEQS_PER_W = BATCH // NW
CHUNK = 16
NCH = ROWS_PER_W // CHUNK
PBLK = 16
LANES = 16
GROUPS = EMBEDDING_DIM // LANES
FACTOR = float(np.sqrt(EMBEDDING_DIM))


def _pos_encoding() -> np.ndarray:
    depth = EMBEDDING_DIM / 2
    positions = np.arange(SEQUENCE_LEN)[:, np.newaxis]
    depths = np.arange(depth)[np.newaxis, :] / depth
    rates = 1 / 10000 ** depths
    radians = positions * rates
    return np.concatenate(
        [np.sin(radians), np.cos(radians)], axis=-1).astype(np.float32)


_PE = _pos_encoding()


def _out_rows() -> np.ndarray:
    w = np.arange(NW)[:, None, None]
    c = np.arange(NCH)[None, :, None]
    j = np.arange(CHUNK)[None, None, :]
    s = (c & 1) * CHUNK + j
    p = c >> 1
    return (w * ROWS_PER_W + s * SEQUENCE_LEN + p).astype(np.int32)


_OUT_ROWS = _out_rows()


def _body(table, idx, pe, orow, out, idx_v, oix_v, pe_v,
          in0, in1, out0, out1, gs0, gs1, ws0, ws1):
    wid = lax.axis_index("s") * NUM_CORES + lax.axis_index("c")
    base = wid * ROWS_PER_W
    pltpu.sync_copy(idx.at[pl.ds(base, ROWS_PER_W)], idx_v)
    pltpu.sync_copy(orow.at[wid], oix_v)

    pltpu.async_copy(table.at[idx_v.at[pl.ds(0, CHUNK)]], in0, gs0)
    pltpu.async_copy(table.at[idx_v.at[pl.ds(CHUNK, CHUNK)]], in1, gs1)

    def step(c, inb, outb, gs, ws):
        pltpu.make_async_copy(table.at[pl.ds(0, CHUNK)], inb, gs).wait()

        @pl.when(c >= 2)
        def _():
            pltpu.make_async_copy(outb, out.at[pl.ds(0, CHUNK)], ws).wait()

        pr = (c >> 1) & (PBLK - 1)

        def grp(g, _):
            sl = pl.ds(g * LANES, LANES)
            peg = pe_v[pr, sl]
            vals = [inb[r, sl] for r in range(CHUNK)]
            for r in range(CHUNK):
                outb[r, sl] = vals[r] * FACTOR + peg
            return 0
        lax.fori_loop(0, GROUPS, grp, 0, unroll=2)

        pltpu.async_copy(outb, out.at[oix_v.at[c]], ws)

        @pl.when(c < NCH - 2)
        def _():
            pltpu.async_copy(
                table.at[idx_v.at[pl.ds((c + 2) * CHUNK, CHUNK)]], inb, gs)

    def c_loop(c, _):
        @pl.when((c & 31) == 0)
        def _():
            pltpu.sync_copy(pe.at[pl.ds((c >> 5) * PBLK, PBLK)], pe_v)

        @pl.when((c & 1) == 0)
        def _():
            step(c, in0, out0, gs0, ws0)

        @pl.when((c & 1) == 1)
        def _():
            step(c, in1, out1, gs1, ws1)
        return 0

    lax.fori_loop(0, NCH, c_loop, 0)

    pltpu.make_async_copy(out0, out.at[pl.ds(0, CHUNK)], ws0).wait()
    pltpu.make_async_copy(out1, out.at[pl.ds(0, CHUNK)], ws1).wait()


@jax.jit
def _embed(encoding, table):
    idx = (encoding.reshape(NW, SEQS_PER_W, SEQUENCE_LEN)
           .transpose(0, 2, 1).reshape(ROWS).astype(jnp.int32))
    pe = jnp.asarray(_PE)
    orow = jnp.asarray(_OUT_ROWS)
    mesh = plsc.VectorSubcoreMesh(core_axis_name="c", subcore_axis_name="s")
    k = pl.kernel(
        _body,
        out_type=jax.ShapeDtypeStruct((ROWS, EMBEDDING_DIM), jnp.float32),
        mesh=mesh,
        scratch_types=[
            pltpu.VMEM((ROWS_PER_W,), jnp.int32),
            pltpu.VMEM((NCH, CHUNK), jnp.int32),
            pltpu.VMEM((PBLK, EMBEDDING_DIM), jnp.float32),
            pltpu.VMEM((CHUNK, EMBEDDING_DIM), jnp.float32),
            pltpu.VMEM((CHUNK, EMBEDDING_DIM), jnp.float32),
            pltpu.VMEM((CHUNK, EMBEDDING_DIM), jnp.float32),
            pltpu.VMEM((CHUNK, EMBEDDING_DIM), jnp.float32),
            pltpu.SemaphoreType.DMA,
            pltpu.SemaphoreType.DMA,
            pltpu.SemaphoreType.DMA,
            pltpu.SemaphoreType.DMA,
        ],
    )
    out = k(table, idx, pe, orow)
    return out.reshape(BATCH, SEQUENCE_LEN, EMBEDDING_DIM)


def kernel(encoding, table):
    return _embed(encoding, table)


# speedup vs baseline: 1.8401x; 1.0917x over previous
import numpy as np
import jax
import jax.numpy as jnp
from jax import lax
from jax.experimental import pallas as pl
from jax.experimental.pallas import tpu as pltpu
from jax.experimental.pallas import tpu_sc as plsc

EMBEDDING_DIM = 1024
SEQUENCE_LEN = 160
VOCAB_SIZE = 100000
BATCH = 1024

NUM_CORES = 2
NUM_SUBCORES = 16
NW = NUM_CORES * NUM_SUBCORES
ROWS = BATCH * SEQUENCE_LEN
ROWS_PER_W = ROWS // NW
SEQS_PER_W = BATCH // NW
CHUNK = 16
NCH = ROWS_PER_W // CHUNK
PBLK = 8
LANES = 16
GROUPS = EMBEDDING_DIM // LANES
FACTOR = float(np.sqrt(EMBEDDING_DIM))
NIN = 3


def _pos_encoding() -> np.ndarray:
    depth = EMBEDDING_DIM / 2
    positions = np.arange(SEQUENCE_LEN)[:, np.newaxis]
    depths = np.arange(depth)[np.newaxis, :] / depth
    rates = 1 / 10000 ** depths
    radians = positions * rates
    return np.concatenate(
        [np.sin(radians), np.cos(radians)], axis=-1).astype(np.float32)


_PE = _pos_encoding()


def _body(table, idx, pe, out, idx_v, pe_v, oix_v,
          in0, in1, in2, out0, out1, gs0, gs1, gs2, ws0, ws1):
    ins, gss = (in0, in1, in2), (gs0, gs1, gs2)
    outs, wss = (out0, out1), (ws0, ws1)
    wid = lax.axis_index("s") * NUM_CORES + lax.axis_index("c")
    base = wid * ROWS_PER_W
    pltpu.sync_copy(idx.at[pl.ds(base, ROWS_PER_W)], idx_v)
    jrow = lax.iota(jnp.int32, 16) * SEQUENCE_LEN

    pltpu.async_copy(table.at[idx_v.at[pl.ds(0, CHUNK)]], in0, gs0)
    pltpu.async_copy(table.at[idx_v.at[pl.ds(CHUNK, CHUNK)]], in1, gs1)

    def step(c, k3, k2):
        inb, gs = ins[k3], gss[k3]
        inp, gsp = ins[(k3 + 2) % NIN], gss[(k3 + 2) % NIN]
        outb, ws = outs[k2], wss[k2]

        @pl.when(c >= 2)
        def _():
            pltpu.make_async_copy(outb, out.at[pl.ds(0, CHUNK)], ws).wait()

        @pl.when(c < NCH - 2)
        def _():
            pltpu.async_copy(
                table.at[idx_v.at[pl.ds((c + 2) * CHUNK, CHUNK)]], inp, gsp)

        pltpu.make_async_copy(table.at[pl.ds(0, CHUNK)], inb, gs).wait()

        pr = (c >> 1) & (PBLK - 1)
        oix_v[k2, :] = jrow + (base + (c & 1) * (CHUNK * SEQUENCE_LEN)
                               + (c >> 1))

        def grp(g, _):
            sl = pl.ds(g * LANES, LANES)
            peg = pe_v[pr, sl]
            vals = [inb[r, sl] for r in range(CHUNK)]
            for r in range(CHUNK):
                outb[r, sl] = vals[r] * FACTOR + peg
            return 0
        lax.fori_loop(0, GROUPS, grp, 0, unroll=2)

        pltpu.async_copy(outb, out.at[oix_v.at[k2]], ws)

    def c_loop(c, c3):
        @pl.when((c & 15) == 0)
        def _():
            pltpu.sync_copy(pe.at[pl.ds((c >> 4) * PBLK, PBLK)], pe_v)

        for k3 in range(NIN):
            for k2 in range(2):
                @pl.when((c3 == k3) & ((c & 1) == k2))
                def _(c=c, k3=k3, k2=k2):
                    step(c, k3, k2)
        return jnp.where(c3 == NIN - 1, 0, c3 + 1)

    lax.fori_loop(0, NCH, c_loop, jnp.int32(0))

    pltpu.make_async_copy(out0, out.at[pl.ds(0, CHUNK)], ws0).wait()
    pltpu.make_async_copy(out1, out.at[pl.ds(0, CHUNK)], ws1).wait()


@jax.jit
def _embed(encoding, table):
    idx = (encoding.reshape(NW, SEQS_PER_W, SEQUENCE_LEN)
           .transpose(0, 2, 1).reshape(ROWS).astype(jnp.int32))
    pe = jnp.asarray(_PE)
    mesh = plsc.VectorSubcoreMesh(core_axis_name="c", subcore_axis_name="s")
    k = pl.kernel(
        _body,
        out_type=jax.ShapeDtypeStruct((ROWS, EMBEDDING_DIM), jnp.float32),
        mesh=mesh,
        scratch_types=[
            pltpu.VMEM((ROWS_PER_W,), jnp.int32),
            pltpu.VMEM((PBLK, EMBEDDING_DIM), jnp.float32),
            pltpu.VMEM((2, CHUNK), jnp.int32),
            pltpu.VMEM((CHUNK, EMBEDDING_DIM), jnp.float32),
            pltpu.VMEM((CHUNK, EMBEDDING_DIM), jnp.float32),
            pltpu.VMEM((CHUNK, EMBEDDING_DIM), jnp.float32),
            pltpu.VMEM((CHUNK, EMBEDDING_DIM), jnp.float32),
            pltpu.VMEM((CHUNK, EMBEDDING_DIM), jnp.float32),
            pltpu.SemaphoreType.DMA,
            pltpu.SemaphoreType.DMA,
            pltpu.SemaphoreType.DMA,
            pltpu.SemaphoreType.DMA,
            pltpu.SemaphoreType.DMA,
        ],
    )
    out = k(table, idx, pe)
    return out.reshape(BATCH, SEQUENCE_LEN, EMBEDDING_DIM)


def kernel(encoding, table):
    return _embed(encoding, table)


# speedup vs baseline: 1.8478x; 1.0042x over previous
import numpy as np
import jax
import jax.numpy as jnp
from jax import lax
from jax.experimental import pallas as pl
from jax.experimental.pallas import tpu as pltpu
from jax.experimental.pallas import tpu_sc as plsc

EMBEDDING_DIM = 1024
SEQUENCE_LEN = 160
VOCAB_SIZE = 100000
BATCH = 1024

NUM_CORES = 2
NUM_SUBCORES = 16
NW = NUM_CORES * NUM_SUBCORES
ROWS = BATCH * SEQUENCE_LEN
ROWS_PER_W = ROWS // NW
SEQS_PER_W = BATCH // NW
CHUNK = 16
NCH = ROWS_PER_W // CHUNK
PBLK = 16
LANES = 16
GROUPS = EMBEDDING_DIM // LANES
FACTOR = float(np.sqrt(EMBEDDING_DIM))
NIN = 3


def _pos_encoding() -> np.ndarray:
    depth = EMBEDDING_DIM / 2
    positions = np.arange(SEQUENCE_LEN)[:, np.newaxis]
    depths = np.arange(depth)[np.newaxis, :] / depth
    rates = 1 / 10000 ** depths
    radians = positions * rates
    return np.concatenate(
        [np.sin(radians), np.cos(radians)], axis=-1).astype(np.float32)


_PE = _pos_encoding()


def _body(table, idx, pe, out, idx_v, pe_v, oix_v,
          in0, in1, in2, out0, out1, gs0, gs1, gs2, ws0, ws1):
    ins, gss = (in0, in1, in2), (gs0, gs1, gs2)
    outs, wss = (out0, out1), (ws0, ws1)
    wid = lax.axis_index("s") * NUM_CORES + lax.axis_index("c")
    base = wid * ROWS_PER_W
    pltpu.sync_copy(idx.at[pl.ds(base, ROWS_PER_W)], idx_v)
    jrow = lax.iota(jnp.int32, 16) * SEQUENCE_LEN

    pltpu.async_copy(table.at[idx_v.at[pl.ds(0, CHUNK)]], in0, gs0)
    pltpu.async_copy(table.at[idx_v.at[pl.ds(CHUNK, CHUNK)]], in1, gs1)

    def step(c, k3, k2):
        inb, gs = ins[k3], gss[k3]
        inp, gsp = ins[(k3 + 2) % NIN], gss[(k3 + 2) % NIN]
        outb, ws = outs[k2], wss[k2]

        @pl.when(c >= 2)
        def _():
            pltpu.make_async_copy(outb, out.at[pl.ds(0, CHUNK)], ws).wait()

        @pl.when(c < NCH - 2)
        def _():
            pltpu.async_copy(
                table.at[idx_v.at[pl.ds((c + 2) * CHUNK, CHUNK)]], inp, gsp)

        pltpu.make_async_copy(table.at[pl.ds(0, CHUNK)], inb, gs).wait()

        pr = (c >> 1) & (PBLK - 1)
        oix_v[k2, :] = jrow + (base + (c & 1) * (CHUNK * SEQUENCE_LEN)
                               + (c >> 1))

        def grp(g, _):
            sl = pl.ds(g * LANES, LANES)
            peg = pe_v[pr, sl]
            vals = [inb[r, sl] for r in range(CHUNK)]
            for r in range(CHUNK):
                outb[r, sl] = vals[r] * FACTOR + peg
            return 0
        lax.fori_loop(0, GROUPS, grp, 0, unroll=2)

        pltpu.async_copy(outb, out.at[oix_v.at[k2]], ws)

    def c_loop(c, c3):
        @pl.when((c & 31) == 0)
        def _():
            pltpu.sync_copy(pe.at[pl.ds((c >> 5) * PBLK, PBLK)], pe_v)

        for k3 in range(NIN):
            for k2 in range(2):
                @pl.when((c3 == k3) & ((c & 1) == k2))
                def _(c=c, k3=k3, k2=k2):
                    step(c, k3, k2)
        return jnp.where(c3 == NIN - 1, 0, c3 + 1)

    lax.fori_loop(0, NCH, c_loop, jnp.int32(0))

    pltpu.make_async_copy(out0, out.at[pl.ds(0, CHUNK)], ws0).wait()
    pltpu.make_async_copy(out1, out.at[pl.ds(0, CHUNK)], ws1).wait()


@jax.jit
def _embed(encoding, table):
    idx = (encoding.reshape(NW, SEQS_PER_W, SEQUENCE_LEN)
           .transpose(0, 2, 1).reshape(ROWS).astype(jnp.int32))
    pe = jnp.asarray(_PE)
    mesh = plsc.VectorSubcoreMesh(core_axis_name="c", subcore_axis_name="s")
    k = pl.kernel(
        _body,
        out_type=jax.ShapeDtypeStruct((ROWS, EMBEDDING_DIM), jnp.float32),
        mesh=mesh,
        scratch_types=[
            pltpu.VMEM((ROWS_PER_W,), jnp.int32),
            pltpu.VMEM((PBLK, EMBEDDING_DIM), jnp.float32),
            pltpu.VMEM((2, CHUNK), jnp.int32),
            pltpu.VMEM((CHUNK, EMBEDDING_DIM), jnp.float32),
            pltpu.VMEM((CHUNK, EMBEDDING_DIM), jnp.float32),
            pltpu.VMEM((CHUNK, EMBEDDING_DIM), jnp.float32),
            pltpu.VMEM((CHUNK, EMBEDDING_DIM), jnp.float32),
            pltpu.VMEM((CHUNK, EMBEDDING_DIM), jnp.float32),
            pltpu.SemaphoreType.DMA,
            pltpu.SemaphoreType.DMA,
            pltpu.SemaphoreType.DMA,
            pltpu.SemaphoreType.DMA,
            pltpu.SemaphoreType.DMA,
        ],
    )
    out = k(table, idx, pe)
    return out.reshape(BATCH, SEQUENCE_LEN, EMBEDDING_DIM)


def kernel(encoding, table):
    return _embed(encoding, table)
